# speedup vs baseline: 2.0603x; 2.0603x over previous
import jax
import jax.numpy as jnp
from jax import lax
from jax.experimental import pallas as pl
from jax.experimental.pallas import tpu as pltpu

_H = 28
_W = 28
_HW = _H * _W


def _inception_kernel(x_ref, w1_ref, b1cm_ref, w23_ref, b23_ref, wblk_ref,
                      bcvcm_ref, w4_ref, b4cm_ref, o_ref, xp23_ref, xp4_ref):
    cin, c1 = w1_ref.shape
    cr = w23_ref.shape[1]
    cc = wblk_ref.shape[-1]

    xb = x_ref[0].astype(jnp.bfloat16)

    y1 = lax.dot_general(w1_ref[...], xb, (((0,), (0,)), ((), ())),
                         preferred_element_type=jnp.float32)
    o_ref[0, 0:c1, :] = jnp.maximum(y1 + b1cm_ref[...], 0.0)

    y23 = lax.dot_general(xb, w23_ref[...], (((0,), (0,)), ((), ())),
                          preferred_element_type=jnp.float32)
    y23 = jnp.maximum(y23 + b23_ref[...], 0.0).astype(jnp.bfloat16)

    xp23_ref[...] = jnp.zeros_like(xp23_ref)
    xp23_ref[1:_H + 1, 1:_W + 1, :] = y23.reshape(_H, _W, cr)
    wblk = wblk_ref[...]
    acc = bcvcm_ref[...]
    for dy in range(3):
        for dx in range(3):
            tap = xp23_ref[dy:dy + _H, dx:dx + _W, :].reshape(_HW, cr)
            acc = acc + lax.dot_general(wblk[dy, dx], tap,
                                        (((0,), (1,)), ((), ())),
                                        preferred_element_type=jnp.float32)
    o_ref[0, c1:c1 + cc, :] = jnp.maximum(acc, 0.0)

    xt = jnp.transpose(xb, (1, 0))
    xp4_ref[...] = jnp.full_like(xp4_ref, -jnp.inf)
    xp4_ref[1:_H + 1, 1:_W + 1, :] = xt.reshape(_H, _W, cin)
    pooled = xp4_ref[1:_H + 1, 1:_W + 1, :]
    for dy in range(3):
        for dx in range(3):
            if dy == 1 and dx == 1:
                continue
            pooled = jnp.maximum(pooled, xp4_ref[dy:dy + _H, dx:dx + _W, :])
    y4 = lax.dot_general(w4_ref[...], pooled.reshape(_HW, cin),
                         (((0,), (1,)), ((), ())),
                         preferred_element_type=jnp.float32)
    o_ref[0, c1 + cc:, :] = jnp.maximum(y4 + b4cm_ref[...], 0.0)


def kernel(x_nchw, b1_w, b2_red_w, b3_red_w, fused1x1_w, fused1x1_bias,
           b2_conv_w, b2_conv_bias, b3_conv_w, b3_conv_bias,
           b4_proj_w, b4_proj_bias):
    n, cin, h, w = x_nchw.shape
    hw = h * w
    c1 = b1_w.shape[1]
    c2r = b2_red_w.shape[1]
    c3r = b3_red_w.shape[1]
    c2 = b2_conv_w.shape[-1]
    c3 = b3_conv_w.shape[-1]
    c4 = b4_proj_w.shape[-1]
    cr = c2r + c3r
    cc = c2 + c3
    cout = c1 + cc + c4

    x = x_nchw.reshape(n, cin, hw)

    w23 = jnp.concatenate([b2_red_w, b3_red_w], axis=1)
    b23 = fused1x1_bias[c1:].reshape(1, cr)
    wblk = jnp.zeros((3, 3, cr, cc), jnp.bfloat16)
    wblk = wblk.at[:, :, :c2r, :c2].set(b2_conv_w)
    wblk = wblk.at[:, :, c2r:, c2:].set(b3_conv_w)
    b1cm = jnp.broadcast_to(fused1x1_bias[:c1, None], (c1, hw))
    bcvcm = jnp.broadcast_to(
        jnp.concatenate([b2_conv_bias, b3_conv_bias])[:, None], (cc, hw))
    b4cm = jnp.broadcast_to(b4_proj_bias[:, None], (c4, hw))

    out = pl.pallas_call(
        _inception_kernel,
        out_shape=jax.ShapeDtypeStruct((n, cout, hw), jnp.float32),
        grid=(n,),
        in_specs=[
            pl.BlockSpec((1, cin, hw), lambda i: (i, 0, 0)),
            pl.BlockSpec((cin, c1), lambda i: (0, 0)),
            pl.BlockSpec((c1, hw), lambda i: (0, 0)),
            pl.BlockSpec((cin, cr), lambda i: (0, 0)),
            pl.BlockSpec((1, cr), lambda i: (0, 0)),
            pl.BlockSpec((3, 3, cr, cc), lambda i: (0, 0, 0, 0)),
            pl.BlockSpec((cc, hw), lambda i: (0, 0)),
            pl.BlockSpec((cin, c4), lambda i: (0, 0)),
            pl.BlockSpec((c4, hw), lambda i: (0, 0)),
        ],
        out_specs=pl.BlockSpec((1, cout, hw), lambda i: (i, 0, 0)),
        scratch_shapes=[pltpu.VMEM((h + 2, w + 2, cr), jnp.bfloat16),
                        pltpu.VMEM((h + 2, w + 2, cin), jnp.bfloat16)],
        compiler_params=pltpu.CompilerParams(
            dimension_semantics=("parallel",),
            vmem_limit_bytes=64 * 1024 * 1024,
        ),
    )(x, b1_w, b1cm, w23, b23, wblk, bcvcm, b4_proj_w, b4cm)
    return out.reshape(n, cout, h, w)


# speedup vs baseline: 2.4932x; 1.2101x over previous
import jax
import jax.numpy as jnp
from jax import lax
from jax.experimental import pallas as pl
from jax.experimental.pallas import tpu as pltpu

_H = 28
_W = 28
_HW = _H * _W
_C00 = (((0,), (0,)), ((), ()))


def _dot(a, b):
    return lax.dot_general(a, b, _C00, preferred_element_type=jnp.float32)


def _inception_kernel(x_ref, w1_ref, b1cm_ref, w23_ref, b23cm_ref, wblk_ref,
                      bcvcm_ref, w4_ref, b4cm_ref, cmul_ref, cadd_ref,
                      fmul_ref, o_ref):
    c1 = w1_ref.shape[1]
    cc = wblk_ref.shape[-1]

    xb = x_ref[0].astype(jnp.bfloat16)

    o_ref[0, 0:c1, :] = jnp.maximum(_dot(w1_ref[...], xb) + b1cm_ref[...],
                                    0.0)

    y23 = jnp.maximum(_dot(w23_ref[...], xb) + b23cm_ref[...], 0.0)
    y23 = y23.astype(jnp.bfloat16)

    tl = pltpu.roll(y23, 1, axis=1) * cmul_ref[0:1, :]
    tr = pltpu.roll(y23, _HW - 1, axis=1) * cmul_ref[1:2, :]
    wblk = wblk_ref[...]
    z = []
    for dy in range(3):
        a = _dot(wblk[dy, 0], tl)
        a = a + _dot(wblk[dy, 1], y23)
        a = a + _dot(wblk[dy, 2], tr)
        z.append(a)
    acc = bcvcm_ref[...] + z[1]
    acc = acc + pltpu.roll(z[0], _W, axis=1) * fmul_ref[0:1, :]
    acc = acc + pltpu.roll(z[2], _HW - _W, axis=1) * fmul_ref[1:2, :]
    o_ref[0, c1:c1 + cc, :] = jnp.maximum(acc, 0.0)

    h = jnp.maximum(pltpu.roll(xb, 1, axis=1) + cadd_ref[0:1, :],
                    pltpu.roll(xb, _HW - 1, axis=1) + cadd_ref[1:2, :])
    h = jnp.maximum(h, xb)
    p2 = jnp.maximum(pltpu.roll(h, _W, axis=1) + cadd_ref[2:3, :],
                     pltpu.roll(h, _HW - _W, axis=1) + cadd_ref[3:4, :])
    p2 = jnp.maximum(p2, h)
    o_ref[0, c1 + cc:, :] = jnp.maximum(_dot(w4_ref[...], p2)
                                        + b4cm_ref[...], 0.0)


def kernel(x_nchw, b1_w, b2_red_w, b3_red_w, fused1x1_w, fused1x1_bias,
           b2_conv_w, b2_conv_bias, b3_conv_w, b3_conv_bias,
           b4_proj_w, b4_proj_bias):
    n, cin, h, w = x_nchw.shape
    hw = h * w
    c1 = b1_w.shape[1]
    c2r = b2_red_w.shape[1]
    c3r = b3_red_w.shape[1]
    c2 = b2_conv_w.shape[-1]
    c3 = b3_conv_w.shape[-1]
    c4 = b4_proj_w.shape[-1]
    cr = c2r + c3r
    cc = c2 + c3
    cout = c1 + cc + c4

    x = x_nchw.reshape(n, cin, hw)

    w23 = jnp.concatenate([b2_red_w, b3_red_w], axis=1)
    wblk = jnp.zeros((3, 3, cr, cc), jnp.bfloat16)
    wblk = wblk.at[:, :, :c2r, :c2].set(b2_conv_w)
    wblk = wblk.at[:, :, c2r:, c2:].set(b3_conv_w)
    b1cm = jnp.broadcast_to(fused1x1_bias[:c1, None], (c1, hw))
    b23cm = jnp.broadcast_to(fused1x1_bias[c1:, None], (cr, hw))
    bcvcm = jnp.broadcast_to(
        jnp.concatenate([b2_conv_bias, b3_conv_bias])[:, None], (cc, hw))
    b4cm = jnp.broadcast_to(b4_proj_bias[:, None], (c4, hw))

    p = jnp.arange(hw)
    col = p % w
    lvalid = col != 0
    rvalid = col != (w - 1)
    tvalid = p >= w
    bvalid = p < (hw - w)
    cmul = jnp.stack([lvalid, rvalid]).astype(jnp.bfloat16)
    ninf = jnp.float32(-jnp.inf)
    cadd = jnp.stack([jnp.where(lvalid, 0.0, ninf),
                      jnp.where(rvalid, 0.0, ninf),
                      jnp.where(tvalid, 0.0, ninf),
                      jnp.where(bvalid, 0.0, ninf)]).astype(jnp.bfloat16)
    fmul = jnp.stack([tvalid, bvalid]).astype(jnp.float32)

    out = pl.pallas_call(
        _inception_kernel,
        out_shape=jax.ShapeDtypeStruct((n, cout, hw), jnp.float32),
        grid=(n,),
        in_specs=[
            pl.BlockSpec((1, cin, hw), lambda i: (i, 0, 0)),
            pl.BlockSpec((cin, c1), lambda i: (0, 0)),
            pl.BlockSpec((c1, hw), lambda i: (0, 0)),
            pl.BlockSpec((cin, cr), lambda i: (0, 0)),
            pl.BlockSpec((cr, hw), lambda i: (0, 0)),
            pl.BlockSpec((3, 3, cr, cc), lambda i: (0, 0, 0, 0)),
            pl.BlockSpec((cc, hw), lambda i: (0, 0)),
            pl.BlockSpec((cin, c4), lambda i: (0, 0)),
            pl.BlockSpec((c4, hw), lambda i: (0, 0)),
            pl.BlockSpec((2, hw), lambda i: (0, 0)),
            pl.BlockSpec((4, hw), lambda i: (0, 0)),
            pl.BlockSpec((2, hw), lambda i: (0, 0)),
        ],
        out_specs=pl.BlockSpec((1, cout, hw), lambda i: (i, 0, 0)),
        compiler_params=pltpu.CompilerParams(
            dimension_semantics=("parallel",),
            vmem_limit_bytes=64 * 1024 * 1024,
        ),
    )(x, b1_w, b1cm, w23, b23cm, wblk, bcvcm, b4_proj_w, b4cm,
      cmul, cadd, fmul)
    return out.reshape(n, cout, h, w)
